# baseline (speedup 1.0000x reference)
import jax
import jax.numpy as jnp
from jax.experimental import pallas as pl
from jax.experimental.pallas import tpu as pltpu

TOKENS = 16384
HIDDEN = 2048
EXPERTS = 64
BLOCK_T = 1024
NBLOCKS = TOKENS // BLOCK_T
NBUF = 3
NSPLIT = 8
SUB = BLOCK_T // NSPLIT


def _router_body(x_hbm, w_ref, b_ref, o_ref, xbuf, sems):
    w = w_ref[...].astype(jnp.bfloat16)
    bias = b_ref[...]

    def copy_sub(block, slot, j):
        return pltpu.make_async_copy(
            x_hbm.at[pl.ds(block * BLOCK_T + j * SUB, SUB), :],
            xbuf.at[slot, pl.ds(j * SUB, SUB), :],
            sems.at[slot, j],
        )

    for slot in range(min(NBUF, NBLOCKS)):
        for j in range(NSPLIT):
            copy_sub(slot, slot, j).start()

    for block in range(NBLOCKS):
        slot = block % NBUF
        for j in range(NSPLIT):
            copy_sub(block, slot, j).wait()
        xb = xbuf[slot].astype(jnp.bfloat16)
        logits = jax.lax.dot_general(
            xb, w,
            dimension_numbers=(((1,), (1,)), ((), ())),
            preferred_element_type=jnp.float32,
        )
        o_ref[pl.ds(block * BLOCK_T, BLOCK_T), :] = jnp.maximum(logits + bias, 0.0)
        nxt = block + NBUF
        if nxt < NBLOCKS:
            for j in range(NSPLIT):
                copy_sub(nxt, slot, j).start()


@jax.jit
def kernel(x, W, b):
    b2 = b.reshape(1, EXPERTS)
    return pl.pallas_call(
        _router_body,
        in_specs=[
            pl.BlockSpec(memory_space=pltpu.MemorySpace.HBM),
            pl.BlockSpec(memory_space=pltpu.MemorySpace.VMEM),
            pl.BlockSpec(memory_space=pltpu.MemorySpace.VMEM),
        ],
        out_specs=pl.BlockSpec(memory_space=pltpu.MemorySpace.VMEM),
        out_shape=jax.ShapeDtypeStruct((TOKENS, EXPERTS), jnp.float32),
        scratch_shapes=[
            pltpu.VMEM((NBUF, BLOCK_T, HIDDEN), jnp.float32),
            pltpu.SemaphoreType.DMA((NBUF, NSPLIT)),
        ],
    )(x, W, b2)


# speedup vs baseline: 1.2026x; 1.2026x over previous
import jax
import jax.numpy as jnp
from jax.experimental import pallas as pl
from jax.experimental.pallas import tpu as pltpu

TOKENS = 16384
HIDDEN = 2048
EXPERTS = 64
BLOCK_T = 1024


def _router_body(x_ref, w_ref, b_ref, o_ref):
    x = x_ref[...].astype(jnp.bfloat16)
    w = w_ref[...].astype(jnp.bfloat16)
    logits = jax.lax.dot_general(
        w, x,
        dimension_numbers=(((1,), (1,)), ((), ())),
        preferred_element_type=jnp.float32,
    )
    o_ref[...] = jnp.maximum(logits + b_ref[...], 0.0)


@jax.jit
def kernel(x, W, b):
    b2 = b.reshape(EXPERTS, 1)
    grid = (TOKENS // BLOCK_T,)
    out_t = pl.pallas_call(
        _router_body,
        grid=grid,
        in_specs=[
            pl.BlockSpec((BLOCK_T, HIDDEN), lambda i: (i, 0)),
            pl.BlockSpec((EXPERTS, HIDDEN), lambda i: (0, 0)),
            pl.BlockSpec((EXPERTS, 1), lambda i: (0, 0)),
        ],
        out_specs=pl.BlockSpec((EXPERTS, BLOCK_T), lambda i: (0, i)),
        out_shape=jax.ShapeDtypeStruct((EXPERTS, TOKENS), jnp.float32),
        compiler_params=pltpu.CompilerParams(
            dimension_semantics=("parallel",),
        ),
    )(x, W, b2)
    return out_t.T
